# baseline (speedup 1.0000x reference)
import functools

import jax
import jax.numpy as jnp
from jax.experimental import pallas as pl
from jax.experimental.pallas import tpu as pltpu

_NUM_CLASSES = 10
_BN_EPS = 1e-5
_LANE = 128
_VMEM_LIMIT = 32 * 1024 * 1024


def _fold_bn(conv_b, gamma, beta, run_mean, run_var):
    inv_std = 1.0 / jnp.sqrt(run_var + _BN_EPS)
    scale = gamma * inv_std
    shift = (conv_b - run_mean) * scale + beta
    return (scale.reshape(1, -1).astype(jnp.float32),
            shift.reshape(1, -1).astype(jnp.float32))


def _tap_major(conv_w):
    cout, cin = conv_w.shape[0], conv_w.shape[1]
    w = jnp.transpose(conv_w, (2, 3, 1, 0)).reshape(9, cin, cout)
    return w.astype(jnp.bfloat16)


def _bn_relu_pool(acc, scale, shift, h, w):
    c = acc.shape[-1]
    y = jnp.maximum(acc * scale + shift, 0.0)
    y = jnp.max(y.reshape(h * (w // 2), 2, c), axis=1)
    y = jnp.max(y.reshape(h // 2, 2, w // 2, c), axis=1)
    return y


def _fused_convs_kernel(xp_ref, w1_ref, s1_ref, t1_ref,
                        w2_ref, s2_ref, t2_ref,
                        w3_ref, s3_ref, t3_ref, o_ref, *, H, W):
    dn = (((0,), (0,)), ((), ()))

    acc_e = jax.lax.dot_general(xp_ref[0], w1_ref[...], dimension_numbers=dn,
                                preferred_element_type=jnp.float32)
    acc_o = jax.lax.dot_general(xp_ref[1], w1_ref[...], dimension_numbers=dn,
                                preferred_element_type=jnp.float32)
    s1, t1 = s1_ref[...], t1_ref[...]
    y = jnp.maximum(jnp.maximum(acc_e * s1 + t1, 0.0),
                    jnp.maximum(acc_o * s1 + t1, 0.0))
    y = jnp.max(y.reshape(2, 2, H // 2, 2, 8, 64), axis=3)
    y1p = jnp.pad(y.astype(jnp.bfloat16),
                  ((0, 0), (0, 0), (1, 1), (1, 1), (0, 0)))

    h2 = H // 2
    accs2 = []
    for g2 in range(2):
        acc = jnp.zeros((h2 * 16, 128), jnp.float32)
        for dy in range(3):
            for dx in range(3):
                e = g2 + dx - 1
                eta, eps = e % 2, (e - e % 2) // 2
                pieces = []
                for s3 in range(2):
                    lam = (s3 + eps) % 2
                    kap = (s3 + eps - lam) // 2
                    pieces.append(y1p[eta, lam, dy:dy + h2,
                                      kap + 1:kap + 9, :])
                a = jnp.stack(pieces, axis=0).reshape(h2 * 16, 64)
                acc = acc + jnp.dot(a, w2_ref[dy * 3 + dx],
                                    preferred_element_type=jnp.float32)
        accs2.append(acc)
    s2, t2 = s2_ref[...], t2_ref[...]
    z = jnp.maximum(jnp.maximum(accs2[0] * s2 + t2, 0.0),
                    jnp.maximum(accs2[1] * s2 + t2, 0.0))
    z = jnp.max(z.reshape(2, h2 // 2, 2, 8, 128), axis=2)
    y2p = jnp.pad(z.astype(jnp.bfloat16),
                  ((0, 0), (1, 1), (1, 1), (0, 0)))

    h3 = H // 4
    accs3 = []
    for g4 in range(2):
        acc = jnp.zeros((h3 * 8, 512), jnp.float32)
        for dy in range(3):
            for dx in range(3):
                e = g4 + dx - 1
                eta, eps = e % 2, (e - e % 2) // 2
                a = y2p[eta, dy:dy + h3, eps + 1:eps + 9, :].reshape(h3 * 8, 128)
                acc = acc + jnp.dot(a, w3_ref[dy * 3 + dx],
                                    preferred_element_type=jnp.float32)
        accs3.append(acc)
    s3_, t3_ = s3_ref[...], t3_ref[...]
    u = jnp.maximum(jnp.maximum(accs3[0] * s3_ + t3_, 0.0),
                    jnp.maximum(accs3[1] * s3_ + t3_, 0.0))
    u = jnp.max(u.reshape(h3 // 2, 2, 8, 512), axis=1)

    g = u.astype(jnp.bfloat16).reshape((h3 // 2) * 8, 512).astype(jnp.float32)
    o_ref[...] = jnp.mean(g, axis=0, keepdims=True)


def _head_kernel(g_ref, w1_ref, b1_ref, w2_ref, b2_ref, o_ref):
    h = jnp.dot(g_ref[...].astype(jnp.bfloat16), w1_ref[...],
                preferred_element_type=jnp.float32) + b1_ref[...]
    h = jnp.maximum(h, 0.0)
    out = jnp.dot(h.astype(jnp.bfloat16), w2_ref[...],
                  preferred_element_type=jnp.float32) + b2_ref[...]
    o_ref[...] = out


def _pack_conv1_input(x_nchw):
    n, _, h, w = x_nchw.shape
    xb = x_nchw.astype(jnp.bfloat16)
    xp = jnp.pad(xb, ((0, 0), (0, 0), (1, 1), (1, 1)))
    taps = [xp[:, :, dy:dy + h, dx:dx + w]
            for dy in range(3) for dx in range(3)]
    pk = jnp.stack(taps, axis=1)
    pk = pk.reshape(n, 27, h, w // 8, 2, 2, 2)
    pk = jnp.transpose(pk, (0, 6, 1, 5, 4, 2, 3))
    pk = pk.reshape(n, 2, 27, h * (w // 2))
    return jnp.pad(pk, ((0, 0), (0, 0), (0, 5), (0, 0)))


def kernel(c1_w, c1_b, c1_gamma, c1_beta, c1_mean, c1_var,
           c2_w, c2_b, c2_gamma, c2_beta, c2_mean, c2_var,
           c3_w, c3_b, c3_gamma, c3_beta, c3_mean, c3_var,
           fc1_w, fc1_b, fc2_w, fc2_b, x_nchw):
    n, _, h, w = x_nchw.shape

    xpk = _pack_conv1_input(x_nchw)

    w1 = jnp.transpose(c1_w, (2, 3, 1, 0)).reshape(27, 64)
    w1 = jnp.pad(w1, ((0, 5), (0, 0))).astype(jnp.bfloat16)
    s1, t1 = _fold_bn(c1_b, c1_gamma, c1_beta, c1_mean, c1_var)
    w2m = _tap_major(c2_w)
    s2, t2 = _fold_bn(c2_b, c2_gamma, c2_beta, c2_mean, c2_var)
    w3m = _tap_major(c3_w)
    s3, t3 = _fold_bn(c3_b, c3_gamma, c3_beta, c3_mean, c3_var)

    body = functools.partial(_fused_convs_kernel, H=h, W=w)

    def conv_chunk(xc):
        nc = xc.shape[0]
        xpk = _pack_conv1_input(xc)
        return pl.pallas_call(
            body,
            out_shape=jax.ShapeDtypeStruct((nc, 1, 512), jnp.float32),
            grid=(nc,),
            in_specs=[
                pl.BlockSpec((None, 2, 32, h * w // 2), lambda i: (i, 0, 0, 0)),
                pl.BlockSpec((32, 64), lambda i: (0, 0)),
                pl.BlockSpec((1, 64), lambda i: (0, 0)),
                pl.BlockSpec((1, 64), lambda i: (0, 0)),
                pl.BlockSpec((9, 64, 128), lambda i: (0, 0, 0)),
                pl.BlockSpec((1, 128), lambda i: (0, 0)),
                pl.BlockSpec((1, 128), lambda i: (0, 0)),
                pl.BlockSpec((9, 128, 512), lambda i: (0, 0, 0)),
                pl.BlockSpec((1, 512), lambda i: (0, 0)),
                pl.BlockSpec((1, 512), lambda i: (0, 0)),
            ],
            out_specs=pl.BlockSpec((None, 1, 512), lambda i: (i, 0, 0)),
            compiler_params=pltpu.CompilerParams(
                dimension_semantics=("arbitrary",),
                vmem_limit_bytes=_VMEM_LIMIT,
            ),
        )(xpk, w1, s1, t1, w2m, s2, t2, w3m, s3, t3)

    n_chunks = 8 if n % 8 == 0 else 1
    cs = n // n_chunks
    gap = jnp.concatenate(
        [conv_chunk(x_nchw[i * cs:(i + 1) * cs]) for i in range(n_chunks)],
        axis=0)
    g = gap.reshape(n, 512)

    w1f = fc1_w.astype(jnp.bfloat16)
    b1f = fc1_b.reshape(1, -1).astype(jnp.float32)
    npad = _LANE
    w2f = jnp.pad(fc2_w, ((0, 0), (0, npad - _NUM_CLASSES))).astype(jnp.bfloat16)
    b2f = jnp.pad(fc2_b, (0, npad - _NUM_CLASSES)).reshape(1, -1).astype(jnp.float32)

    logits = pl.pallas_call(
        _head_kernel,
        out_shape=jax.ShapeDtypeStruct((n, npad), jnp.float32),
        grid=(1,),
        in_specs=[
            pl.BlockSpec((n, 512), lambda i: (0, 0)),
            pl.BlockSpec((512, 1024), lambda i: (0, 0)),
            pl.BlockSpec((1, 1024), lambda i: (0, 0)),
            pl.BlockSpec((1024, npad), lambda i: (0, 0)),
            pl.BlockSpec((1, npad), lambda i: (0, 0)),
        ],
        out_specs=pl.BlockSpec((n, npad), lambda i: (0, 0)),
        compiler_params=pltpu.CompilerParams(
            dimension_semantics=("arbitrary",),
            vmem_limit_bytes=_VMEM_LIMIT,
        ),
    )(g, w1f, b1f, w2f, b2f)
    return logits[:, :_NUM_CLASSES]


# speedup vs baseline: 1.0356x; 1.0356x over previous
import functools

import jax
import jax.numpy as jnp
from jax.experimental import pallas as pl
from jax.experimental.pallas import tpu as pltpu

_NUM_CLASSES = 10
_BN_EPS = 1e-5
_LANE = 128
_VMEM_LIMIT = 32 * 1024 * 1024


def _fold_bn(conv_b, gamma, beta, run_mean, run_var):
    inv_std = 1.0 / jnp.sqrt(run_var + _BN_EPS)
    scale = gamma * inv_std
    shift = (conv_b - run_mean) * scale + beta
    return (scale.reshape(1, -1).astype(jnp.float32),
            shift.reshape(1, -1).astype(jnp.float32))


def _tap_major(conv_w):
    cout, cin = conv_w.shape[0], conv_w.shape[1]
    w = jnp.transpose(conv_w, (2, 3, 1, 0)).reshape(9, cin, cout)
    return w.astype(jnp.bfloat16)


def _bn_relu_pool(acc, scale, shift, h, w):
    c = acc.shape[-1]
    y = jnp.maximum(acc * scale + shift, 0.0)
    y = jnp.max(y.reshape(h * (w // 2), 2, c), axis=1)
    y = jnp.max(y.reshape(h // 2, 2, w // 2, c), axis=1)
    return y


def _fused_convs_kernel(xp_ref, w1_ref, s1_ref, t1_ref,
                        w2_ref, s2_ref, t2_ref,
                        w3_ref, s3_ref, t3_ref, o_ref, *, H, W):
    dn = (((0,), (0,)), ((), ()))

    acc_e = jax.lax.dot_general(xp_ref[0], w1_ref[...], dimension_numbers=dn,
                                preferred_element_type=jnp.float32)
    acc_o = jax.lax.dot_general(xp_ref[1], w1_ref[...], dimension_numbers=dn,
                                preferred_element_type=jnp.float32)
    s1, t1 = s1_ref[...], t1_ref[...]
    y = jnp.maximum(jnp.maximum(acc_e * s1 + t1, 0.0),
                    jnp.maximum(acc_o * s1 + t1, 0.0))
    y = jnp.max(y.reshape(2, 2, H // 2, 2, 8, 64), axis=3)
    y1p = jnp.pad(y.astype(jnp.bfloat16),
                  ((0, 0), (0, 0), (1, 1), (1, 1), (0, 0)))

    h2 = H // 2
    accs2 = []
    for g2 in range(2):
        acc = jnp.zeros((h2 * 16, 128), jnp.float32)
        for dy in range(3):
            for dx in range(3):
                e = g2 + dx - 1
                eta, eps = e % 2, (e - e % 2) // 2
                pieces = []
                for s3 in range(2):
                    lam = (s3 + eps) % 2
                    kap = (s3 + eps - lam) // 2
                    pieces.append(y1p[eta, lam, dy:dy + h2,
                                      kap + 1:kap + 9, :])
                a = jnp.stack(pieces, axis=0).reshape(h2 * 16, 64)
                acc = acc + jnp.dot(a, w2_ref[dy * 3 + dx],
                                    preferred_element_type=jnp.float32)
        accs2.append(acc)
    s2, t2 = s2_ref[...], t2_ref[...]
    z = jnp.maximum(jnp.maximum(accs2[0] * s2 + t2, 0.0),
                    jnp.maximum(accs2[1] * s2 + t2, 0.0))
    z = jnp.max(z.reshape(2, h2 // 2, 2, 8, 128), axis=2)
    y2p = jnp.pad(z.astype(jnp.bfloat16),
                  ((0, 0), (1, 1), (1, 1), (0, 0)))

    h3 = H // 4
    accs3 = []
    for g4 in range(2):
        acc = jnp.zeros((h3 * 8, 512), jnp.float32)
        for dy in range(3):
            for dx in range(3):
                e = g4 + dx - 1
                eta, eps = e % 2, (e - e % 2) // 2
                a = y2p[eta, dy:dy + h3, eps + 1:eps + 9, :].reshape(h3 * 8, 128)
                acc = acc + jnp.dot(a, w3_ref[dy * 3 + dx],
                                    preferred_element_type=jnp.float32)
        accs3.append(acc)
    s3_, t3_ = s3_ref[...], t3_ref[...]
    u = jnp.maximum(jnp.maximum(accs3[0] * s3_ + t3_, 0.0),
                    jnp.maximum(accs3[1] * s3_ + t3_, 0.0))
    u = jnp.max(u.reshape(h3 // 2, 2, 8, 512), axis=1)

    g = u.astype(jnp.bfloat16).reshape((h3 // 2) * 8, 512).astype(jnp.float32)
    o_ref[...] = jnp.mean(g, axis=0, keepdims=True)


def _head_kernel(g_ref, w1_ref, b1_ref, w2_ref, b2_ref, o_ref):
    h = jnp.dot(g_ref[...].astype(jnp.bfloat16), w1_ref[...],
                preferred_element_type=jnp.float32) + b1_ref[...]
    h = jnp.maximum(h, 0.0)
    out = jnp.dot(h.astype(jnp.bfloat16), w2_ref[...],
                  preferred_element_type=jnp.float32) + b2_ref[...]
    o_ref[...] = out


def _pack_conv1_input(x_nchw):
    n, _, h, w = x_nchw.shape
    xb = x_nchw.astype(jnp.bfloat16)
    xp = jnp.pad(xb, ((0, 0), (0, 0), (1, 1), (1, 1)))
    taps = [xp[:, :, dy:dy + h, dx:dx + w]
            for dy in range(3) for dx in range(3)]
    pk = jnp.stack(taps, axis=1)
    pk = pk.reshape(n, 27, h, w // 8, 2, 2, 2)
    pk = jnp.transpose(pk, (0, 6, 1, 5, 4, 2, 3))
    pk = pk.reshape(n, 2, 27, h * (w // 2))
    return jnp.pad(pk, ((0, 0), (0, 0), (0, 5), (0, 0)))


def kernel(c1_w, c1_b, c1_gamma, c1_beta, c1_mean, c1_var,
           c2_w, c2_b, c2_gamma, c2_beta, c2_mean, c2_var,
           c3_w, c3_b, c3_gamma, c3_beta, c3_mean, c3_var,
           fc1_w, fc1_b, fc2_w, fc2_b, x_nchw):
    n, _, h, w = x_nchw.shape

    xpk = _pack_conv1_input(x_nchw)

    w1 = jnp.transpose(c1_w, (2, 3, 1, 0)).reshape(27, 64)
    w1 = jnp.pad(w1, ((0, 5), (0, 0))).astype(jnp.bfloat16)
    s1, t1 = _fold_bn(c1_b, c1_gamma, c1_beta, c1_mean, c1_var)
    w2m = _tap_major(c2_w)
    s2, t2 = _fold_bn(c2_b, c2_gamma, c2_beta, c2_mean, c2_var)
    w3m = _tap_major(c3_w)
    s3, t3 = _fold_bn(c3_b, c3_gamma, c3_beta, c3_mean, c3_var)

    body = functools.partial(_fused_convs_kernel, H=h, W=w)

    def conv_chunk(xc):
        nc = xc.shape[0]
        xpk = _pack_conv1_input(xc)
        return pl.pallas_call(
            body,
            out_shape=jax.ShapeDtypeStruct((nc, 1, 512), jnp.float32),
            grid=(nc,),
            in_specs=[
                pl.BlockSpec((None, 2, 32, h * w // 2), lambda i: (i, 0, 0, 0)),
                pl.BlockSpec((32, 64), lambda i: (0, 0)),
                pl.BlockSpec((1, 64), lambda i: (0, 0)),
                pl.BlockSpec((1, 64), lambda i: (0, 0)),
                pl.BlockSpec((9, 64, 128), lambda i: (0, 0, 0)),
                pl.BlockSpec((1, 128), lambda i: (0, 0)),
                pl.BlockSpec((1, 128), lambda i: (0, 0)),
                pl.BlockSpec((9, 128, 512), lambda i: (0, 0, 0)),
                pl.BlockSpec((1, 512), lambda i: (0, 0)),
                pl.BlockSpec((1, 512), lambda i: (0, 0)),
            ],
            out_specs=pl.BlockSpec((None, 1, 512), lambda i: (i, 0, 0)),
            compiler_params=pltpu.CompilerParams(
                dimension_semantics=("arbitrary",),
                vmem_limit_bytes=_VMEM_LIMIT,
            ),
        )(xpk, w1, s1, t1, w2m, s2, t2, w3m, s3, t3)

    n_chunks = 6 if n % 6 == 0 else (4 if n % 4 == 0 else 1)
    cs = n // n_chunks
    gap = jnp.concatenate(
        [conv_chunk(x_nchw[i * cs:(i + 1) * cs]) for i in range(n_chunks)],
        axis=0)
    g = gap.reshape(n, 512)

    w1f = fc1_w.astype(jnp.bfloat16)
    b1f = fc1_b.reshape(1, -1).astype(jnp.float32)
    npad = _LANE
    w2f = jnp.pad(fc2_w, ((0, 0), (0, npad - _NUM_CLASSES))).astype(jnp.bfloat16)
    b2f = jnp.pad(fc2_b, (0, npad - _NUM_CLASSES)).reshape(1, -1).astype(jnp.float32)

    logits = pl.pallas_call(
        _head_kernel,
        out_shape=jax.ShapeDtypeStruct((n, npad), jnp.float32),
        grid=(1,),
        in_specs=[
            pl.BlockSpec((n, 512), lambda i: (0, 0)),
            pl.BlockSpec((512, 1024), lambda i: (0, 0)),
            pl.BlockSpec((1, 1024), lambda i: (0, 0)),
            pl.BlockSpec((1024, npad), lambda i: (0, 0)),
            pl.BlockSpec((1, npad), lambda i: (0, 0)),
        ],
        out_specs=pl.BlockSpec((n, npad), lambda i: (0, 0)),
        compiler_params=pltpu.CompilerParams(
            dimension_semantics=("arbitrary",),
            vmem_limit_bytes=_VMEM_LIMIT,
        ),
    )(g, w1f, b1f, w2f, b2f)
    return logits[:, :_NUM_CLASSES]


# speedup vs baseline: 1.0604x; 1.0240x over previous
import functools

import jax
import jax.numpy as jnp
from jax.experimental import pallas as pl
from jax.experimental.pallas import tpu as pltpu

_NUM_CLASSES = 10
_BN_EPS = 1e-5
_LANE = 128
_VMEM_LIMIT = 32 * 1024 * 1024


def _fold_bn(conv_b, gamma, beta, run_mean, run_var):
    inv_std = 1.0 / jnp.sqrt(run_var + _BN_EPS)
    scale = gamma * inv_std
    shift = (conv_b - run_mean) * scale + beta
    return (scale.reshape(1, -1).astype(jnp.float32),
            shift.reshape(1, -1).astype(jnp.float32))


def _tap_major(conv_w):
    cout, cin = conv_w.shape[0], conv_w.shape[1]
    w = jnp.transpose(conv_w, (2, 3, 1, 0)).reshape(9, cin, cout)
    return w.astype(jnp.bfloat16)


def _bn_relu_pool(acc, scale, shift, h, w):
    c = acc.shape[-1]
    y = jnp.maximum(acc * scale + shift, 0.0)
    y = jnp.max(y.reshape(h * (w // 2), 2, c), axis=1)
    y = jnp.max(y.reshape(h // 2, 2, w // 2, c), axis=1)
    return y


def _fused_convs_kernel(xp_ref, w1_ref, s1_ref, t1_ref,
                        w2_ref, s2_ref, t2_ref,
                        w3_ref, s3_ref, t3_ref, o_ref, *, H, W, B):
    dn = (((0,), (0,)), ((), ()))
    for img in range(B):
        _one_image(xp_ref[img], w1_ref, s1_ref, t1_ref, w2_ref, s2_ref, t2_ref,
                   w3_ref, s3_ref, t3_ref, o_ref, img, H, dn)


def _one_image(xp, w1_ref, s1_ref, t1_ref, w2_ref, s2_ref, t2_ref,
               w3_ref, s3_ref, t3_ref, o_ref, img, H, dn):
    acc_e = jax.lax.dot_general(xp[0], w1_ref[...], dimension_numbers=dn,
                                preferred_element_type=jnp.float32)
    acc_o = jax.lax.dot_general(xp[1], w1_ref[...], dimension_numbers=dn,
                                preferred_element_type=jnp.float32)
    s1, t1 = s1_ref[...], t1_ref[...]
    y = jnp.maximum(jnp.maximum(acc_e * s1 + t1, 0.0),
                    jnp.maximum(acc_o * s1 + t1, 0.0))
    y = jnp.max(y.reshape(2, 2, H // 2, 2, 8, 64), axis=3)
    y1p = jnp.pad(y.astype(jnp.bfloat16),
                  ((0, 0), (0, 0), (1, 1), (1, 1), (0, 0)))

    h2 = H // 2
    accs2 = []
    for g2 in range(2):
        acc = jnp.zeros((h2 * 16, 128), jnp.float32)
        for dy in range(3):
            for dx in range(3):
                e = g2 + dx - 1
                eta, eps = e % 2, (e - e % 2) // 2
                pieces = []
                for s3 in range(2):
                    lam = (s3 + eps) % 2
                    kap = (s3 + eps - lam) // 2
                    pieces.append(y1p[eta, lam, dy:dy + h2,
                                      kap + 1:kap + 9, :])
                a = jnp.stack(pieces, axis=0).reshape(h2 * 16, 64)
                acc = acc + jnp.dot(a, w2_ref[dy * 3 + dx],
                                    preferred_element_type=jnp.float32)
        accs2.append(acc)
    s2, t2 = s2_ref[...], t2_ref[...]
    z = jnp.maximum(jnp.maximum(accs2[0] * s2 + t2, 0.0),
                    jnp.maximum(accs2[1] * s2 + t2, 0.0))
    z = jnp.max(z.reshape(2, h2 // 2, 2, 8, 128), axis=2)
    y2p = jnp.pad(z.astype(jnp.bfloat16),
                  ((0, 0), (1, 1), (1, 1), (0, 0)))

    h3 = H // 4
    accs3 = []
    for g4 in range(2):
        acc = jnp.zeros((h3 * 8, 512), jnp.float32)
        for dy in range(3):
            for dx in range(3):
                e = g4 + dx - 1
                eta, eps = e % 2, (e - e % 2) // 2
                a = y2p[eta, dy:dy + h3, eps + 1:eps + 9, :].reshape(h3 * 8, 128)
                acc = acc + jnp.dot(a, w3_ref[dy * 3 + dx],
                                    preferred_element_type=jnp.float32)
        accs3.append(acc)
    s3_, t3_ = s3_ref[...], t3_ref[...]
    u = jnp.maximum(jnp.maximum(accs3[0] * s3_ + t3_, 0.0),
                    jnp.maximum(accs3[1] * s3_ + t3_, 0.0))
    u = jnp.max(u.reshape(h3 // 2, 2, 8, 512), axis=1)

    g = u.astype(jnp.bfloat16).reshape((h3 // 2) * 8, 512).astype(jnp.float32)
    o_ref[img] = jnp.mean(g, axis=0, keepdims=True)


def _head_kernel(g_ref, w1_ref, b1_ref, w2_ref, b2_ref, o_ref):
    h = jnp.dot(g_ref[...].astype(jnp.bfloat16), w1_ref[...],
                preferred_element_type=jnp.float32) + b1_ref[...]
    h = jnp.maximum(h, 0.0)
    out = jnp.dot(h.astype(jnp.bfloat16), w2_ref[...],
                  preferred_element_type=jnp.float32) + b2_ref[...]
    o_ref[...] = out


def _pack_conv1_input(x_nchw):
    n, _, h, w = x_nchw.shape
    xb = x_nchw.astype(jnp.bfloat16)
    xp = jnp.pad(xb, ((0, 0), (0, 0), (1, 1), (1, 1)))
    taps = [xp[:, :, dy:dy + h, dx:dx + w]
            for dy in range(3) for dx in range(3)]
    pk = jnp.stack(taps, axis=1)
    pk = pk.reshape(n, 27, h, w // 8, 2, 2, 2)
    pk = jnp.transpose(pk, (0, 6, 1, 5, 4, 2, 3))
    pk = pk.reshape(n, 2, 27, h * (w // 2))
    return jnp.pad(pk, ((0, 0), (0, 0), (0, 5), (0, 0)))


def kernel(c1_w, c1_b, c1_gamma, c1_beta, c1_mean, c1_var,
           c2_w, c2_b, c2_gamma, c2_beta, c2_mean, c2_var,
           c3_w, c3_b, c3_gamma, c3_beta, c3_mean, c3_var,
           fc1_w, fc1_b, fc2_w, fc2_b, x_nchw):
    n, _, h, w = x_nchw.shape

    xpk = _pack_conv1_input(x_nchw)

    w1 = jnp.transpose(c1_w, (2, 3, 1, 0)).reshape(27, 64)
    w1 = jnp.pad(w1, ((0, 5), (0, 0))).astype(jnp.bfloat16)
    s1, t1 = _fold_bn(c1_b, c1_gamma, c1_beta, c1_mean, c1_var)
    w2m = _tap_major(c2_w)
    s2, t2 = _fold_bn(c2_b, c2_gamma, c2_beta, c2_mean, c2_var)
    w3m = _tap_major(c3_w)
    s3, t3 = _fold_bn(c3_b, c3_gamma, c3_beta, c3_mean, c3_var)

    img_block = 2
    body = functools.partial(_fused_convs_kernel, H=h, W=w, B=img_block)

    def conv_chunk(xc):
        nc = xc.shape[0]
        xpk = _pack_conv1_input(xc)
        return pl.pallas_call(
            body,
            out_shape=jax.ShapeDtypeStruct((nc, 1, 512), jnp.float32),
            grid=(nc // img_block,),
            in_specs=[
                pl.BlockSpec((img_block, 2, 32, h * w // 2),
                             lambda i: (i, 0, 0, 0)),
                pl.BlockSpec((32, 64), lambda i: (0, 0)),
                pl.BlockSpec((1, 64), lambda i: (0, 0)),
                pl.BlockSpec((1, 64), lambda i: (0, 0)),
                pl.BlockSpec((9, 64, 128), lambda i: (0, 0, 0)),
                pl.BlockSpec((1, 128), lambda i: (0, 0)),
                pl.BlockSpec((1, 128), lambda i: (0, 0)),
                pl.BlockSpec((9, 128, 512), lambda i: (0, 0, 0)),
                pl.BlockSpec((1, 512), lambda i: (0, 0)),
                pl.BlockSpec((1, 512), lambda i: (0, 0)),
            ],
            out_specs=pl.BlockSpec((img_block, 1, 512), lambda i: (i, 0, 0)),
            compiler_params=pltpu.CompilerParams(
                dimension_semantics=("arbitrary",),
                vmem_limit_bytes=_VMEM_LIMIT,
            ),
        )(xpk, w1, s1, t1, w2m, s2, t2, w3m, s3, t3)

    n_chunks = 6 if n % 6 == 0 else (4 if n % 4 == 0 else 1)
    cs = n // n_chunks
    gap = jnp.concatenate(
        [conv_chunk(x_nchw[i * cs:(i + 1) * cs]) for i in range(n_chunks)],
        axis=0)
    g = gap.reshape(n, 512)

    w1f = fc1_w.astype(jnp.bfloat16)
    b1f = fc1_b.reshape(1, -1).astype(jnp.float32)
    npad = _LANE
    w2f = jnp.pad(fc2_w, ((0, 0), (0, npad - _NUM_CLASSES))).astype(jnp.bfloat16)
    b2f = jnp.pad(fc2_b, (0, npad - _NUM_CLASSES)).reshape(1, -1).astype(jnp.float32)

    logits = pl.pallas_call(
        _head_kernel,
        out_shape=jax.ShapeDtypeStruct((n, npad), jnp.float32),
        grid=(1,),
        in_specs=[
            pl.BlockSpec((n, 512), lambda i: (0, 0)),
            pl.BlockSpec((512, 1024), lambda i: (0, 0)),
            pl.BlockSpec((1, 1024), lambda i: (0, 0)),
            pl.BlockSpec((1024, npad), lambda i: (0, 0)),
            pl.BlockSpec((1, npad), lambda i: (0, 0)),
        ],
        out_specs=pl.BlockSpec((n, npad), lambda i: (0, 0)),
        compiler_params=pltpu.CompilerParams(
            dimension_semantics=("arbitrary",),
            vmem_limit_bytes=_VMEM_LIMIT,
        ),
    )(g, w1f, b1f, w2f, b2f)
    return logits[:, :_NUM_CLASSES]


# speedup vs baseline: 1.0747x; 1.0135x over previous
import functools

import jax
import jax.numpy as jnp
from jax.experimental import pallas as pl
from jax.experimental.pallas import tpu as pltpu

_NUM_CLASSES = 10
_BN_EPS = 1e-5
_LANE = 128
_VMEM_LIMIT = 32 * 1024 * 1024


def _fold_bn(conv_b, gamma, beta, run_mean, run_var):
    inv_std = 1.0 / jnp.sqrt(run_var + _BN_EPS)
    scale = gamma * inv_std
    shift = (conv_b - run_mean) * scale + beta
    return (scale.reshape(1, -1).astype(jnp.float32),
            shift.reshape(1, -1).astype(jnp.float32))


def _tap_major(conv_w):
    cout, cin = conv_w.shape[0], conv_w.shape[1]
    w = jnp.transpose(conv_w, (2, 3, 1, 0)).reshape(9, cin, cout)
    return w.astype(jnp.bfloat16)


def _bn_relu_pool(acc, scale, shift, h, w):
    c = acc.shape[-1]
    y = jnp.maximum(acc * scale + shift, 0.0)
    y = jnp.max(y.reshape(h * (w // 2), 2, c), axis=1)
    y = jnp.max(y.reshape(h // 2, 2, w // 2, c), axis=1)
    return y


def _fused_convs_kernel(xp_ref, w1_ref, s1_ref, t1_ref,
                        w2_ref, s2_ref, t2_ref,
                        w3_ref, s3_ref, t3_ref, o_ref, *, H, W, B):
    dn = (((0,), (0,)), ((), ()))
    for img in range(B):
        _one_image(xp_ref[img], w1_ref, s1_ref, t1_ref, w2_ref, s2_ref, t2_ref,
                   w3_ref, s3_ref, t3_ref, o_ref, img, H, dn)


def _one_image(xp, w1_ref, s1_ref, t1_ref, w2_ref, s2_ref, t2_ref,
               w3_ref, s3_ref, t3_ref, o_ref, img, H, dn):
    acc_e = jax.lax.dot_general(xp[0], w1_ref[...], dimension_numbers=dn,
                                preferred_element_type=jnp.float32)
    acc_o = jax.lax.dot_general(xp[1], w1_ref[...], dimension_numbers=dn,
                                preferred_element_type=jnp.float32)
    s1, t1 = s1_ref[...], t1_ref[...]
    y = jnp.maximum(jnp.maximum(acc_e * s1 + t1, 0.0),
                    jnp.maximum(acc_o * s1 + t1, 0.0))
    y = jnp.max(y.reshape(2, 2, H // 2, 2, 8, 64), axis=3)
    y1p = jnp.pad(y.astype(jnp.bfloat16),
                  ((0, 0), (0, 0), (1, 1), (1, 1), (0, 0)))

    h2 = H // 2
    accs2 = []
    for g2 in range(2):
        acc = jnp.zeros((h2 * 16, 128), jnp.float32)
        for dy in range(3):
            for dx in range(3):
                e = g2 + dx - 1
                eta, eps = e % 2, (e - e % 2) // 2
                pieces = []
                for s3 in range(2):
                    lam = (s3 + eps) % 2
                    kap = (s3 + eps - lam) // 2
                    pieces.append(y1p[eta, lam, dy:dy + h2,
                                      kap + 1:kap + 9, :])
                a = jnp.stack(pieces, axis=0).reshape(h2 * 16, 64)
                acc = acc + jnp.dot(a, w2_ref[dy * 3 + dx],
                                    preferred_element_type=jnp.float32)
        accs2.append(acc)
    s2, t2 = s2_ref[...], t2_ref[...]
    z = jnp.maximum(jnp.maximum(accs2[0] * s2 + t2, 0.0),
                    jnp.maximum(accs2[1] * s2 + t2, 0.0))
    z = jnp.max(z.reshape(2, h2 // 2, 2, 8, 128), axis=2)
    y2p = jnp.pad(z.astype(jnp.bfloat16),
                  ((0, 0), (1, 1), (1, 1), (0, 0)))

    h3 = H // 4
    accs3 = []
    for g4 in range(2):
        acc = jnp.zeros((h3 * 8, 512), jnp.float32)
        for dy in range(3):
            for dx in range(3):
                e = g4 + dx - 1
                eta, eps = e % 2, (e - e % 2) // 2
                a = y2p[eta, dy:dy + h3, eps + 1:eps + 9, :].reshape(h3 * 8, 128)
                acc = acc + jnp.dot(a, w3_ref[dy * 3 + dx],
                                    preferred_element_type=jnp.float32)
        accs3.append(acc)
    s3_, t3_ = s3_ref[...], t3_ref[...]
    u = jnp.maximum(jnp.maximum(accs3[0] * s3_ + t3_, 0.0),
                    jnp.maximum(accs3[1] * s3_ + t3_, 0.0))
    u = jnp.max(u.reshape(h3 // 2, 2, 8, 512), axis=1)

    g = u.astype(jnp.bfloat16).reshape((h3 // 2) * 8, 512).astype(jnp.float32)
    o_ref[img] = jnp.mean(g, axis=0, keepdims=True)


def _head_kernel(g_ref, w1_ref, b1_ref, w2_ref, b2_ref, o_ref):
    h = jnp.dot(g_ref[...].astype(jnp.bfloat16), w1_ref[...],
                preferred_element_type=jnp.float32) + b1_ref[...]
    h = jnp.maximum(h, 0.0)
    out = jnp.dot(h.astype(jnp.bfloat16), w2_ref[...],
                  preferred_element_type=jnp.float32) + b2_ref[...]
    o_ref[...] = out


def _pack_conv1_input(x_nchw):
    n, _, h, w = x_nchw.shape
    xb = x_nchw.astype(jnp.bfloat16)
    xp = jnp.pad(xb, ((0, 0), (0, 0), (1, 1), (1, 1)))
    taps = [xp[:, :, dy:dy + h, dx:dx + w]
            for dy in range(3) for dx in range(3)]
    pk = jnp.stack(taps, axis=1)
    pk = pk.reshape(n, 27, h, w // 8, 2, 2, 2)
    pk = jnp.transpose(pk, (0, 6, 1, 5, 4, 2, 3))
    pk = pk.reshape(n, 2, 27, h * (w // 2))
    return jnp.pad(pk, ((0, 0), (0, 0), (0, 5), (0, 0)))


def kernel(c1_w, c1_b, c1_gamma, c1_beta, c1_mean, c1_var,
           c2_w, c2_b, c2_gamma, c2_beta, c2_mean, c2_var,
           c3_w, c3_b, c3_gamma, c3_beta, c3_mean, c3_var,
           fc1_w, fc1_b, fc2_w, fc2_b, x_nchw):
    n, _, h, w = x_nchw.shape

    xpk = _pack_conv1_input(x_nchw)

    w1 = jnp.transpose(c1_w, (2, 3, 1, 0)).reshape(27, 64)
    w1 = jnp.pad(w1, ((0, 5), (0, 0))).astype(jnp.bfloat16)
    s1, t1 = _fold_bn(c1_b, c1_gamma, c1_beta, c1_mean, c1_var)
    w2m = _tap_major(c2_w)
    s2, t2 = _fold_bn(c2_b, c2_gamma, c2_beta, c2_mean, c2_var)
    w3m = _tap_major(c3_w)
    s3, t3 = _fold_bn(c3_b, c3_gamma, c3_beta, c3_mean, c3_var)

    img_block = 4
    body = functools.partial(_fused_convs_kernel, H=h, W=w, B=img_block)

    def conv_chunk(xc):
        nc = xc.shape[0]
        xpk = _pack_conv1_input(xc)
        return pl.pallas_call(
            body,
            out_shape=jax.ShapeDtypeStruct((nc, 1, 512), jnp.float32),
            grid=(nc // img_block,),
            in_specs=[
                pl.BlockSpec((img_block, 2, 32, h * w // 2),
                             lambda i: (i, 0, 0, 0)),
                pl.BlockSpec((32, 64), lambda i: (0, 0)),
                pl.BlockSpec((1, 64), lambda i: (0, 0)),
                pl.BlockSpec((1, 64), lambda i: (0, 0)),
                pl.BlockSpec((9, 64, 128), lambda i: (0, 0, 0)),
                pl.BlockSpec((1, 128), lambda i: (0, 0)),
                pl.BlockSpec((1, 128), lambda i: (0, 0)),
                pl.BlockSpec((9, 128, 512), lambda i: (0, 0, 0)),
                pl.BlockSpec((1, 512), lambda i: (0, 0)),
                pl.BlockSpec((1, 512), lambda i: (0, 0)),
            ],
            out_specs=pl.BlockSpec((img_block, 1, 512), lambda i: (i, 0, 0)),
            compiler_params=pltpu.CompilerParams(
                dimension_semantics=("arbitrary",),
                vmem_limit_bytes=_VMEM_LIMIT,
            ),
        )(xpk, w1, s1, t1, w2m, s2, t2, w3m, s3, t3)

    n_chunks = 6 if n % 6 == 0 else (4 if n % 4 == 0 else 1)
    cs = n // n_chunks
    gap = jnp.concatenate(
        [conv_chunk(x_nchw[i * cs:(i + 1) * cs]) for i in range(n_chunks)],
        axis=0)
    g = gap.reshape(n, 512)

    w1f = fc1_w.astype(jnp.bfloat16)
    b1f = fc1_b.reshape(1, -1).astype(jnp.float32)
    npad = _LANE
    w2f = jnp.pad(fc2_w, ((0, 0), (0, npad - _NUM_CLASSES))).astype(jnp.bfloat16)
    b2f = jnp.pad(fc2_b, (0, npad - _NUM_CLASSES)).reshape(1, -1).astype(jnp.float32)

    logits = pl.pallas_call(
        _head_kernel,
        out_shape=jax.ShapeDtypeStruct((n, npad), jnp.float32),
        grid=(1,),
        in_specs=[
            pl.BlockSpec((n, 512), lambda i: (0, 0)),
            pl.BlockSpec((512, 1024), lambda i: (0, 0)),
            pl.BlockSpec((1, 1024), lambda i: (0, 0)),
            pl.BlockSpec((1024, npad), lambda i: (0, 0)),
            pl.BlockSpec((1, npad), lambda i: (0, 0)),
        ],
        out_specs=pl.BlockSpec((n, npad), lambda i: (0, 0)),
        compiler_params=pltpu.CompilerParams(
            dimension_semantics=("arbitrary",),
            vmem_limit_bytes=_VMEM_LIMIT,
        ),
    )(g, w1f, b1f, w2f, b2f)
    return logits[:, :_NUM_CLASSES]


# speedup vs baseline: 1.0855x; 1.0100x over previous
import functools

import jax
import jax.numpy as jnp
from jax.experimental import pallas as pl
from jax.experimental.pallas import tpu as pltpu

_NUM_CLASSES = 10
_BN_EPS = 1e-5
_LANE = 128
_VMEM_LIMIT = 32 * 1024 * 1024


def _fold_bn(conv_b, gamma, beta, run_mean, run_var):
    inv_std = 1.0 / jnp.sqrt(run_var + _BN_EPS)
    scale = gamma * inv_std
    shift = (conv_b - run_mean) * scale + beta
    return (scale.reshape(1, -1).astype(jnp.float32),
            shift.reshape(1, -1).astype(jnp.float32))


def _tap_major(conv_w):
    cout, cin = conv_w.shape[0], conv_w.shape[1]
    w = jnp.transpose(conv_w, (2, 3, 1, 0)).reshape(9, cin, cout)
    return w.astype(jnp.bfloat16)


def _bn_relu_pool(acc, scale, shift, h, w):
    c = acc.shape[-1]
    y = jnp.maximum(acc * scale + shift, 0.0)
    y = jnp.max(y.reshape(h * (w // 2), 2, c), axis=1)
    y = jnp.max(y.reshape(h // 2, 2, w // 2, c), axis=1)
    return y


def _fused_convs_kernel(xp_ref, w1_ref, s1_ref, t1_ref,
                        w2_ref, s2_ref, t2_ref,
                        w3_ref, s3_ref, t3_ref, o_ref, *, H, W, B):
    dn = (((0,), (0,)), ((), ()))
    for img in range(B):
        _one_image(xp_ref[img], w1_ref, s1_ref, t1_ref, w2_ref, s2_ref, t2_ref,
                   w3_ref, s3_ref, t3_ref, o_ref, img, H, dn)


def _one_image(xp, w1_ref, s1_ref, t1_ref, w2_ref, s2_ref, t2_ref,
               w3_ref, s3_ref, t3_ref, o_ref, img, H, dn):
    acc_e = jax.lax.dot_general(xp[0], w1_ref[...], dimension_numbers=dn,
                                preferred_element_type=jnp.float32)
    acc_o = jax.lax.dot_general(xp[1], w1_ref[...], dimension_numbers=dn,
                                preferred_element_type=jnp.float32)
    s1, t1 = s1_ref[...], t1_ref[...]
    y = jnp.maximum(jnp.maximum(acc_e * s1 + t1, 0.0),
                    jnp.maximum(acc_o * s1 + t1, 0.0))
    y = jnp.max(y.reshape(2, 2, H // 2, 2, 8, 64), axis=3)
    y1p = jnp.pad(y.astype(jnp.bfloat16),
                  ((0, 0), (0, 0), (1, 1), (1, 1), (0, 0)))

    h2 = H // 2
    accs2 = []
    for g2 in range(2):
        acc = jnp.zeros((h2 * 16, 128), jnp.float32)
        for dy in range(3):
            for dx in range(3):
                e = g2 + dx - 1
                eta, eps = e % 2, (e - e % 2) // 2
                pieces = []
                for s3 in range(2):
                    lam = (s3 + eps) % 2
                    kap = (s3 + eps - lam) // 2
                    pieces.append(y1p[eta, lam, dy:dy + h2,
                                      kap + 1:kap + 9, :])
                a = jnp.stack(pieces, axis=0).reshape(h2 * 16, 64)
                acc = acc + jnp.dot(a, w2_ref[dy * 3 + dx],
                                    preferred_element_type=jnp.float32)
        accs2.append(acc)
    s2, t2 = s2_ref[...], t2_ref[...]
    z = jnp.maximum(jnp.maximum(accs2[0] * s2 + t2, 0.0),
                    jnp.maximum(accs2[1] * s2 + t2, 0.0))
    z = jnp.max(z.reshape(2, h2 // 2, 2, 8, 128), axis=2)
    y2p = jnp.pad(z.astype(jnp.bfloat16),
                  ((0, 0), (1, 1), (1, 1), (0, 0)))

    h3 = H // 4
    accs3 = []
    for g4 in range(2):
        acc = jnp.zeros((h3 * 8, 512), jnp.float32)
        for dy in range(3):
            for dx in range(3):
                e = g4 + dx - 1
                eta, eps = e % 2, (e - e % 2) // 2
                a = y2p[eta, dy:dy + h3, eps + 1:eps + 9, :].reshape(h3 * 8, 128)
                acc = acc + jnp.dot(a, w3_ref[dy * 3 + dx],
                                    preferred_element_type=jnp.float32)
        accs3.append(acc)
    s3_, t3_ = s3_ref[...], t3_ref[...]
    u = jnp.maximum(jnp.maximum(accs3[0] * s3_ + t3_, 0.0),
                    jnp.maximum(accs3[1] * s3_ + t3_, 0.0))
    u = jnp.max(u.reshape(h3 // 2, 2, 8, 512), axis=1)

    g = u.astype(jnp.bfloat16).reshape((h3 // 2) * 8, 512).astype(jnp.float32)
    o_ref[img] = jnp.mean(g, axis=0, keepdims=True)


def _head_kernel(g_ref, w1_ref, b1_ref, w2_ref, b2_ref, o_ref):
    h = jnp.dot(g_ref[...].astype(jnp.bfloat16), w1_ref[...],
                preferred_element_type=jnp.float32) + b1_ref[...]
    h = jnp.maximum(h, 0.0)
    out = jnp.dot(h.astype(jnp.bfloat16), w2_ref[...],
                  preferred_element_type=jnp.float32) + b2_ref[...]
    o_ref[...] = out


def _pack_conv1_input(x_nchw):
    n, _, h, w = x_nchw.shape
    xb = x_nchw.astype(jnp.bfloat16)
    xp = jnp.pad(xb, ((0, 0), (0, 0), (1, 1), (1, 1)))
    taps = [xp[:, :, dy:dy + h, dx:dx + w]
            for dy in range(3) for dx in range(3)]
    pk = jnp.stack(taps, axis=1)
    pk = pk.reshape(n, 27, h, w // 8, 2, 2, 2)
    pk = jnp.transpose(pk, (0, 6, 1, 5, 4, 2, 3))
    pk = pk.reshape(n, 2, 27, h * (w // 2))
    return jnp.pad(pk, ((0, 0), (0, 0), (0, 5), (0, 0)))


def kernel(c1_w, c1_b, c1_gamma, c1_beta, c1_mean, c1_var,
           c2_w, c2_b, c2_gamma, c2_beta, c2_mean, c2_var,
           c3_w, c3_b, c3_gamma, c3_beta, c3_mean, c3_var,
           fc1_w, fc1_b, fc2_w, fc2_b, x_nchw):
    n, _, h, w = x_nchw.shape

    xpk = _pack_conv1_input(x_nchw)

    w1 = jnp.transpose(c1_w, (2, 3, 1, 0)).reshape(27, 64)
    w1 = jnp.pad(w1, ((0, 5), (0, 0))).astype(jnp.bfloat16)
    s1, t1 = _fold_bn(c1_b, c1_gamma, c1_beta, c1_mean, c1_var)
    w2m = _tap_major(c2_w)
    s2, t2 = _fold_bn(c2_b, c2_gamma, c2_beta, c2_mean, c2_var)
    w3m = _tap_major(c3_w)
    s3, t3 = _fold_bn(c3_b, c3_gamma, c3_beta, c3_mean, c3_var)

    img_block = 8
    body = functools.partial(_fused_convs_kernel, H=h, W=w, B=img_block)

    def conv_chunk(xc):
        nc = xc.shape[0]
        xpk = _pack_conv1_input(xc)
        return pl.pallas_call(
            body,
            out_shape=jax.ShapeDtypeStruct((nc, 1, 512), jnp.float32),
            grid=(nc // img_block,),
            in_specs=[
                pl.BlockSpec((img_block, 2, 32, h * w // 2),
                             lambda i: (i, 0, 0, 0)),
                pl.BlockSpec((32, 64), lambda i: (0, 0)),
                pl.BlockSpec((1, 64), lambda i: (0, 0)),
                pl.BlockSpec((1, 64), lambda i: (0, 0)),
                pl.BlockSpec((9, 64, 128), lambda i: (0, 0, 0)),
                pl.BlockSpec((1, 128), lambda i: (0, 0)),
                pl.BlockSpec((1, 128), lambda i: (0, 0)),
                pl.BlockSpec((9, 128, 512), lambda i: (0, 0, 0)),
                pl.BlockSpec((1, 512), lambda i: (0, 0)),
                pl.BlockSpec((1, 512), lambda i: (0, 0)),
            ],
            out_specs=pl.BlockSpec((img_block, 1, 512), lambda i: (i, 0, 0)),
            compiler_params=pltpu.CompilerParams(
                dimension_semantics=("arbitrary",),
                vmem_limit_bytes=_VMEM_LIMIT,
            ),
        )(xpk, w1, s1, t1, w2m, s2, t2, w3m, s3, t3)

    n_chunks = 6 if n % 6 == 0 else (4 if n % 4 == 0 else 1)
    cs = n // n_chunks
    gap = jnp.concatenate(
        [conv_chunk(x_nchw[i * cs:(i + 1) * cs]) for i in range(n_chunks)],
        axis=0)
    g = gap.reshape(n, 512)

    w1f = fc1_w.astype(jnp.bfloat16)
    b1f = fc1_b.reshape(1, -1).astype(jnp.float32)
    npad = _LANE
    w2f = jnp.pad(fc2_w, ((0, 0), (0, npad - _NUM_CLASSES))).astype(jnp.bfloat16)
    b2f = jnp.pad(fc2_b, (0, npad - _NUM_CLASSES)).reshape(1, -1).astype(jnp.float32)

    logits = pl.pallas_call(
        _head_kernel,
        out_shape=jax.ShapeDtypeStruct((n, npad), jnp.float32),
        grid=(1,),
        in_specs=[
            pl.BlockSpec((n, 512), lambda i: (0, 0)),
            pl.BlockSpec((512, 1024), lambda i: (0, 0)),
            pl.BlockSpec((1, 1024), lambda i: (0, 0)),
            pl.BlockSpec((1024, npad), lambda i: (0, 0)),
            pl.BlockSpec((1, npad), lambda i: (0, 0)),
        ],
        out_specs=pl.BlockSpec((n, npad), lambda i: (0, 0)),
        compiler_params=pltpu.CompilerParams(
            dimension_semantics=("arbitrary",),
            vmem_limit_bytes=_VMEM_LIMIT,
        ),
    )(g, w1f, b1f, w2f, b2f)
    return logits[:, :_NUM_CLASSES]


# speedup vs baseline: 1.0861x; 1.0005x over previous
import functools

import jax
import jax.numpy as jnp
from jax.experimental import pallas as pl
from jax.experimental.pallas import tpu as pltpu

_NUM_CLASSES = 10
_BN_EPS = 1e-5
_LANE = 128
_VMEM_LIMIT = 32 * 1024 * 1024


def _fold_bn(conv_b, gamma, beta, run_mean, run_var):
    inv_std = 1.0 / jnp.sqrt(run_var + _BN_EPS)
    scale = gamma * inv_std
    shift = (conv_b - run_mean) * scale + beta
    return (scale.reshape(1, -1).astype(jnp.float32),
            shift.reshape(1, -1).astype(jnp.float32))


def _tap_major(conv_w):
    cout, cin = conv_w.shape[0], conv_w.shape[1]
    w = jnp.transpose(conv_w, (2, 3, 1, 0)).reshape(9, cin, cout)
    return w.astype(jnp.bfloat16)


def _fused_convs_kernel(xp_ref, w1_ref, s1_ref, t1_ref,
                        w2_ref, s2_ref, t2_ref,
                        w3_ref, s3_ref, t3_ref, o_ref, *, H, W, B):
    dn = (((0,), (0,)), ((), ()))
    for img in range(B):
        _one_image(xp_ref[img], w1_ref, s1_ref, t1_ref, w2_ref, s2_ref, t2_ref,
                   w3_ref, s3_ref, t3_ref, o_ref, img, H, dn)


def _one_image(xp, w1_ref, s1_ref, t1_ref, w2_ref, s2_ref, t2_ref,
               w3_ref, s3_ref, t3_ref, o_ref, img, H, dn):
    acc_e = jax.lax.dot_general(xp[0], w1_ref[...], dimension_numbers=dn,
                                preferred_element_type=jnp.float32)
    acc_o = jax.lax.dot_general(xp[1], w1_ref[...], dimension_numbers=dn,
                                preferred_element_type=jnp.float32)
    s1, t1 = s1_ref[...], t1_ref[...]
    y = jnp.maximum(jnp.maximum(acc_e * s1 + t1, 0.0),
                    jnp.maximum(acc_o * s1 + t1, 0.0))
    y = jnp.max(y.reshape(2, 2, H // 2, 2, 8, 64), axis=3)
    y1p = jnp.pad(y.astype(jnp.bfloat16),
                  ((0, 0), (0, 0), (1, 1), (1, 1), (0, 0)))

    h2 = H // 2
    accs2 = []
    for g2 in range(2):
        acc = jnp.zeros((h2 * 16, 128), jnp.float32)
        for dy in range(3):
            for dx in range(3):
                e = g2 + dx - 1
                eta, eps = e % 2, (e - e % 2) // 2
                pieces = []
                for s3 in range(2):
                    lam = (s3 + eps) % 2
                    kap = (s3 + eps - lam) // 2
                    pieces.append(y1p[eta, lam, dy:dy + h2,
                                      kap + 1:kap + 9, :])
                a = jnp.stack(pieces, axis=0).reshape(h2 * 16, 64)
                acc = acc + jnp.dot(a, w2_ref[dy * 3 + dx],
                                    preferred_element_type=jnp.float32)
        accs2.append(acc)
    s2, t2 = s2_ref[...], t2_ref[...]
    z = jnp.maximum(jnp.maximum(accs2[0] * s2 + t2, 0.0),
                    jnp.maximum(accs2[1] * s2 + t2, 0.0))
    z = jnp.max(z.reshape(2, h2 // 2, 2, 8, 128), axis=2)
    y2p = jnp.pad(z.astype(jnp.bfloat16),
                  ((0, 0), (1, 1), (1, 1), (0, 0)))

    h3 = H // 4
    accs3 = []
    for g4 in range(2):
        acc = jnp.zeros((h3 * 8, 512), jnp.float32)
        for dy in range(3):
            for dx in range(3):
                e = g4 + dx - 1
                eta, eps = e % 2, (e - e % 2) // 2
                a = y2p[eta, dy:dy + h3, eps + 1:eps + 9, :].reshape(h3 * 8, 128)
                acc = acc + jnp.dot(a, w3_ref[dy * 3 + dx],
                                    preferred_element_type=jnp.float32)
        accs3.append(acc)
    s3_, t3_ = s3_ref[...], t3_ref[...]
    u = jnp.maximum(jnp.maximum(accs3[0] * s3_ + t3_, 0.0),
                    jnp.maximum(accs3[1] * s3_ + t3_, 0.0))
    u = jnp.max(u.reshape(h3 // 2, 2, 8, 512), axis=1)

    g = u.astype(jnp.bfloat16).reshape((h3 // 2) * 8, 512).astype(jnp.float32)
    o_ref[img] = jnp.mean(g, axis=0, keepdims=True)


def _head_kernel(g_ref, w1_ref, b1_ref, w2_ref, b2_ref, o_ref):
    h = jnp.dot(g_ref[...].astype(jnp.bfloat16), w1_ref[...],
                preferred_element_type=jnp.float32) + b1_ref[...]
    h = jnp.maximum(h, 0.0)
    out = jnp.dot(h.astype(jnp.bfloat16), w2_ref[...],
                  preferred_element_type=jnp.float32) + b2_ref[...]
    o_ref[...] = out


def _pack_conv1_input(x_nchw):
    n, _, h, w = x_nchw.shape
    xb = x_nchw.astype(jnp.bfloat16)
    xp = jnp.pad(xb, ((0, 0), (0, 0), (1, 1), (1, 1)))
    taps = [xp[:, :, dy:dy + h, dx:dx + w]
            for dy in range(3) for dx in range(3)]
    pk = jnp.stack(taps, axis=1)
    pk = pk.reshape(n, 27, h, w // 8, 2, 2, 2)
    pk = jnp.transpose(pk, (0, 6, 1, 5, 4, 2, 3))
    pk = pk.reshape(n, 2, 27, h * (w // 2))
    return jnp.pad(pk, ((0, 0), (0, 0), (0, 5), (0, 0)))


def kernel(c1_w, c1_b, c1_gamma, c1_beta, c1_mean, c1_var,
           c2_w, c2_b, c2_gamma, c2_beta, c2_mean, c2_var,
           c3_w, c3_b, c3_gamma, c3_beta, c3_mean, c3_var,
           fc1_w, fc1_b, fc2_w, fc2_b, x_nchw):
    n, _, h, w = x_nchw.shape

    xpk = _pack_conv1_input(x_nchw)

    w1 = jnp.transpose(c1_w, (2, 3, 1, 0)).reshape(27, 64)
    w1 = jnp.pad(w1, ((0, 5), (0, 0))).astype(jnp.bfloat16)
    s1, t1 = _fold_bn(c1_b, c1_gamma, c1_beta, c1_mean, c1_var)
    w2m = _tap_major(c2_w)
    s2, t2 = _fold_bn(c2_b, c2_gamma, c2_beta, c2_mean, c2_var)
    w3m = _tap_major(c3_w)
    s3, t3 = _fold_bn(c3_b, c3_gamma, c3_beta, c3_mean, c3_var)

    n_chunks = 6 if n % 6 == 0 else (4 if n % 4 == 0 else 1)
    cs = n // n_chunks
    img_block = 8 if cs % 8 == 0 else (2 if cs % 2 == 0 else 1)
    body = functools.partial(_fused_convs_kernel, H=h, W=w, B=img_block)

    def conv_chunk(xc):
        nc = xc.shape[0]
        xpk = _pack_conv1_input(xc)
        return pl.pallas_call(
            body,
            out_shape=jax.ShapeDtypeStruct((nc, 1, 512), jnp.float32),
            grid=(nc // img_block,),
            in_specs=[
                pl.BlockSpec((img_block, 2, 32, h * w // 2),
                             lambda i: (i, 0, 0, 0)),
                pl.BlockSpec((32, 64), lambda i: (0, 0)),
                pl.BlockSpec((1, 64), lambda i: (0, 0)),
                pl.BlockSpec((1, 64), lambda i: (0, 0)),
                pl.BlockSpec((9, 64, 128), lambda i: (0, 0, 0)),
                pl.BlockSpec((1, 128), lambda i: (0, 0)),
                pl.BlockSpec((1, 128), lambda i: (0, 0)),
                pl.BlockSpec((9, 128, 512), lambda i: (0, 0, 0)),
                pl.BlockSpec((1, 512), lambda i: (0, 0)),
                pl.BlockSpec((1, 512), lambda i: (0, 0)),
            ],
            out_specs=pl.BlockSpec((img_block, 1, 512), lambda i: (i, 0, 0)),
            compiler_params=pltpu.CompilerParams(
                dimension_semantics=("arbitrary",),
                vmem_limit_bytes=_VMEM_LIMIT,
            ),
        )(xpk, w1, s1, t1, w2m, s2, t2, w3m, s3, t3)

    gap = jnp.concatenate(
        [conv_chunk(x_nchw[i * cs:(i + 1) * cs]) for i in range(n_chunks)],
        axis=0)
    g = gap.reshape(n, 512)

    w1f = fc1_w.astype(jnp.bfloat16)
    b1f = fc1_b.reshape(1, -1).astype(jnp.float32)
    npad = _LANE
    w2f = jnp.pad(fc2_w, ((0, 0), (0, npad - _NUM_CLASSES))).astype(jnp.bfloat16)
    b2f = jnp.pad(fc2_b, (0, npad - _NUM_CLASSES)).reshape(1, -1).astype(jnp.float32)

    logits = pl.pallas_call(
        _head_kernel,
        out_shape=jax.ShapeDtypeStruct((n, npad), jnp.float32),
        grid=(1,),
        in_specs=[
            pl.BlockSpec((n, 512), lambda i: (0, 0)),
            pl.BlockSpec((512, 1024), lambda i: (0, 0)),
            pl.BlockSpec((1, 1024), lambda i: (0, 0)),
            pl.BlockSpec((1024, npad), lambda i: (0, 0)),
            pl.BlockSpec((1, npad), lambda i: (0, 0)),
        ],
        out_specs=pl.BlockSpec((n, npad), lambda i: (0, 0)),
        compiler_params=pltpu.CompilerParams(
            dimension_semantics=("arbitrary",),
            vmem_limit_bytes=_VMEM_LIMIT,
        ),
    )(g, w1f, b1f, w2f, b2f)
    return logits[:, :_NUM_CLASSES]
